# baseline (speedup 1.0000x reference)
import jax
import jax.numpy as jnp
from jax import lax
from jax.experimental import pallas as pl
from jax.experimental.pallas import tpu as pltpu

B, N, D, H, K = 16, 2048, 768, 256, 5
THRESH = 0.5
KPAD = 8


def _fused_body(mean_ref, max_ref, w1_ref, b1_ref, w2_ref, b2_ref, w3_ref,
                b3_ref, u1_ref, ub1_ref, u2_ref, ub2_ref,
                pud_ref, masks_ref, attn_ref, emb_hbm,
                score_out, vals_out, idx_out, probs_out, flag_out, conf_out,
                attn_out,
                rows_v, idx_v, idx_s, sem_idx, sem_rows):
    f32 = jnp.float32
    i32 = jnp.int32

    pud = pud_ref[...]
    masks = masks_ref[...]
    attn = attn_ref[...]
    masked = jnp.where(masks > 0.5, pud, 0.0)
    iota2 = lax.broadcasted_iota(i32, (B, N), 1)

    vals_cols, idx_cols, div_cols, attn_cols, copies = [], [], [], [], []
    for r in range(K):
        rowmax = jnp.max(masked, axis=1, keepdims=True)
        cand = jnp.where(masked == rowmax, iota2, N)
        rowidx = jnp.min(cand, axis=1, keepdims=True)
        onehot = iota2 == rowidx
        div_cols.append(jnp.sum(jnp.where(onehot, pud, 0.0), axis=1,
                                keepdims=True))
        attn_cols.append(jnp.sum(jnp.where(onehot, attn, 0.0), axis=1,
                                 keepdims=True))
        vals_cols.append(rowmax)
        idx_cols.append(rowidx)
        masked = jnp.where(onehot, -1.0, masked)
    vals5 = jnp.concatenate(vals_cols, axis=1)
    idx5 = jnp.concatenate(idx_cols, axis=1)
    attn5 = jnp.concatenate(attn_cols, axis=1)

    idx_v[...] = jnp.concatenate(
        idx_cols + [idx_cols[-1]] * (KPAD - K), axis=1)
    idx_copy = pltpu.make_async_copy(idx_v, idx_s, sem_idx)
    idx_copy.start()
    idx_copy.wait()

    for j in range(K):
        for b in range(B):
            c = pltpu.make_async_copy(
                emb_hbm.at[b, idx_s[b, j]], rows_v.at[j * B + b], sem_rows)
            c.start()
            copies.append(c)

    meanv = mean_ref[...]
    maxv = max_ref[...]
    w1 = w1_ref[...]
    const_row = 3.0 * w1[2:3, :] + 0.01 * w1[3:4, :] + b1_ref[...]
    h1 = jnp.maximum(meanv * w1[0:1, :] + maxv * w1[1:2, :] + const_row, 0.0)
    h2 = jnp.maximum(
        jnp.dot(h1, w2_ref[...], preferred_element_type=f32) + b2_ref[...],
        0.0)
    logit = jnp.dot(h2, w3_ref[...], preferred_element_type=f32) + b3_ref[...]
    score_out[...] = 1.0 / (1.0 + jnp.exp(-logit))

    for c in copies:
        c.wait()

    u1a = u1_ref[pl.ds(0, D), :]
    u1row = u1_ref[pl.ds(D, 1), :]
    u2 = u2_ref[...]
    u2d = u2[:, 1:2] - u2[:, 0:1]
    ub2 = ub2_ref[...]
    ub2d = ub2[:, 1:2] - ub2[:, 0:1]
    divT = jnp.concatenate(div_cols, axis=0)
    emball = rows_v[...]
    uh = jnp.maximum(
        jnp.dot(emball, u1a, preferred_element_type=f32)
        + divT * u1row + ub1_ref[...], 0.0)
    ld = jnp.dot(uh, u2d, preferred_element_type=f32) + ub2d
    pcol = 1.0 / (1.0 + jnp.exp(-ld))
    probs = jnp.concatenate(
        [lax.slice(pcol, (j * B, 0), ((j + 1) * B, 1)) for j in range(K)],
        axis=1)
    probs_out[...] = probs

    flagb = vals5 > THRESH
    flag_out[...] = flagb
    flagf = flagb.astype(f32)
    cnt = jnp.sum(flagf, axis=1, keepdims=True)
    sumpf = jnp.sum(probs * flagf, axis=1, keepdims=True)
    avg_conf = jnp.where(cnt > 0, sumpf / jnp.maximum(cnt, 1.0), 0.5)
    sep = (maxv - meanv) / (maxv + 1e-8)
    conf_out[...] = jnp.clip(0.4 * sep + 0.15 + 0.3 * avg_conf, 0.0, 1.0)

    vals_out[...] = vals5
    idx_out[...] = idx5
    attn_out[...] = attn5


_fused = pl.pallas_call(
    _fused_body,
    in_specs=[pl.BlockSpec(memory_space=pltpu.MemorySpace.HBM) if i == 15
              else pl.BlockSpec(memory_space=pltpu.MemorySpace.VMEM)
              for i in range(16)],
    out_shape=[
        jax.ShapeDtypeStruct((B, 1), jnp.float32),
        jax.ShapeDtypeStruct((B, K), jnp.float32),
        jax.ShapeDtypeStruct((B, K), jnp.int32),
        jax.ShapeDtypeStruct((B, K), jnp.float32),
        jax.ShapeDtypeStruct((B, K), jnp.bool_),
        jax.ShapeDtypeStruct((B, 1), jnp.float32),
        jax.ShapeDtypeStruct((B, K), jnp.float32),
    ],
    scratch_shapes=[
        pltpu.VMEM((K * B, D), jnp.float32),
        pltpu.VMEM((B, KPAD), jnp.int32),
        pltpu.SMEM((B, KPAD), jnp.int32),
        pltpu.SemaphoreType.DMA,
        pltpu.SemaphoreType.DMA,
    ],
)


def kernel(mean_divergence, max_divergence, per_unit_divergence,
           local_embeddings, unit_masks, attention_weights,
           W1, b1, W2, b2, W3, b3, U1, ub1, U2, ub2):
    score, topk_vals, topk_idx, probs, flag, conf, attn_topk = _fused(
        mean_divergence.reshape(B, 1), max_divergence.reshape(B, 1),
        W1, b1.reshape(1, H), W2, b2.reshape(1, H // 2), W3, b3.reshape(1, 1),
        U1, ub1.reshape(1, H), U2, ub2.reshape(1, 2),
        per_unit_divergence, unit_masks, attention_weights, local_embeddings)

    return (score.reshape(B), topk_vals, topk_idx, probs, flag,
            conf.reshape(B), attn_topk)


# speedup vs baseline: 1.0032x; 1.0032x over previous
import jax
import jax.numpy as jnp
from jax import lax
from jax.experimental import pallas as pl
from jax.experimental.pallas import tpu as pltpu

B, N, D, H, K = 16, 2048, 768, 256, 5
THRESH = 0.5
KPAD = 8


def _fused_body(mean_ref, max_ref, w1_ref, b1_ref, w2_ref, b2_ref, w3_ref,
                b3_ref, u1_ref, ub1_ref, u2_ref, ub2_ref,
                pud_ref, masks_ref, attn_ref, emb_hbm,
                score_out, vals_out, idx_out, probs_out, flag_out, conf_out,
                attn_out,
                rows_v, idx_v, idx_s, sem_idx, sem_rows):
    f32 = jnp.float32
    i32 = jnp.int32

    pud = pud_ref[...]
    masks = masks_ref[...]
    attn = attn_ref[...]
    masked = jnp.where(masks > 0.5, pud, 0.0)
    iota2 = lax.broadcasted_iota(i32, (B, N), 1)

    div_cols, flag_cols, copies = [], [], []
    for r in range(K):
        rowmax = jnp.max(masked, axis=1, keepdims=True)
        cand = jnp.where(masked == rowmax, iota2, N)
        rowidx = jnp.min(cand, axis=1, keepdims=True)
        onehot = iota2 == rowidx
        div_cols.append(jnp.sum(jnp.where(onehot, pud, 0.0), axis=1,
                                keepdims=True))
        attn_out[:, r:r + 1] = jnp.sum(jnp.where(onehot, attn, 0.0), axis=1,
                                       keepdims=True)
        vals_out[:, r:r + 1] = rowmax
        idx_out[:, r:r + 1] = rowidx
        idx_v[:, r:r + 1] = rowidx
        flagb_col = rowmax > THRESH
        flag_out[:, r:r + 1] = flagb_col
        flag_cols.append(flagb_col.astype(f32))
        masked = jnp.where(onehot, -1.0, masked)

    idx_copy = pltpu.make_async_copy(idx_v, idx_s, sem_idx)
    idx_copy.start()
    idx_copy.wait()

    for j in range(K):
        for b in range(B):
            c = pltpu.make_async_copy(
                emb_hbm.at[b, idx_s[b, j]], rows_v.at[j * B + b], sem_rows)
            c.start()
            copies.append(c)

    meanv = mean_ref[...]
    maxv = max_ref[...]
    w1 = w1_ref[...]
    const_row = 3.0 * w1[2:3, :] + 0.01 * w1[3:4, :] + b1_ref[...]
    h1 = jnp.maximum(meanv * w1[0:1, :] + maxv * w1[1:2, :] + const_row, 0.0)
    h2 = jnp.maximum(
        jnp.dot(h1, w2_ref[...], preferred_element_type=f32) + b2_ref[...],
        0.0)
    logit = jnp.dot(h2, w3_ref[...], preferred_element_type=f32) + b3_ref[...]
    score_out[...] = 1.0 / (1.0 + jnp.exp(-logit))

    for c in copies:
        c.wait()

    u1a = u1_ref[pl.ds(0, D), :]
    u1row = u1_ref[pl.ds(D, 1), :]
    u2 = u2_ref[...]
    u2d = u2[:, 1:2] - u2[:, 0:1]
    ub2 = ub2_ref[...]
    ub2d = ub2[:, 1:2] - ub2[:, 0:1]
    divT = jnp.concatenate(div_cols, axis=0)
    emball = rows_v[...]
    uh = jnp.maximum(
        jnp.dot(emball, u1a, preferred_element_type=f32)
        + divT * u1row + ub1_ref[...], 0.0)
    ld = jnp.dot(uh, u2d, preferred_element_type=f32) + ub2d
    pcol = 1.0 / (1.0 + jnp.exp(-ld))
    cnt = flag_cols[0]
    for fc in flag_cols[1:]:
        cnt = cnt + fc
    sumpf = jnp.zeros((B, 1), f32)
    for j in range(K):
        pj = lax.slice(pcol, (j * B, 0), ((j + 1) * B, 1))
        probs_out[:, j:j + 1] = pj
        sumpf = sumpf + pj * flag_cols[j]
    avg_conf = jnp.where(cnt > 0, sumpf / jnp.maximum(cnt, 1.0), 0.5)
    sep = (maxv - meanv) / (maxv + 1e-8)
    conf_out[...] = jnp.clip(0.4 * sep + 0.15 + 0.3 * avg_conf, 0.0, 1.0)


_fused = pl.pallas_call(
    _fused_body,
    in_specs=[pl.BlockSpec(memory_space=pltpu.MemorySpace.HBM) if i == 15
              else pl.BlockSpec(memory_space=pltpu.MemorySpace.VMEM)
              for i in range(16)],
    out_shape=[
        jax.ShapeDtypeStruct((B, 1), jnp.float32),
        jax.ShapeDtypeStruct((B, K), jnp.float32),
        jax.ShapeDtypeStruct((B, K), jnp.int32),
        jax.ShapeDtypeStruct((B, K), jnp.float32),
        jax.ShapeDtypeStruct((B, K), jnp.bool_),
        jax.ShapeDtypeStruct((B, 1), jnp.float32),
        jax.ShapeDtypeStruct((B, K), jnp.float32),
    ],
    scratch_shapes=[
        pltpu.VMEM((K * B, D), jnp.float32),
        pltpu.VMEM((B, KPAD), jnp.int32),
        pltpu.SMEM((B, KPAD), jnp.int32),
        pltpu.SemaphoreType.DMA,
        pltpu.SemaphoreType.DMA,
    ],
)


def kernel(mean_divergence, max_divergence, per_unit_divergence,
           local_embeddings, unit_masks, attention_weights,
           W1, b1, W2, b2, W3, b3, U1, ub1, U2, ub2):
    score, topk_vals, topk_idx, probs, flag, conf, attn_topk = _fused(
        mean_divergence.reshape(B, 1), max_divergence.reshape(B, 1),
        W1, b1.reshape(1, H), W2, b2.reshape(1, H // 2), W3, b3.reshape(1, 1),
        U1, ub1.reshape(1, H), U2, ub2.reshape(1, 2),
        per_unit_divergence, unit_masks, attention_weights, local_embeddings)

    return (score.reshape(B), topk_vals, topk_idx, probs, flag,
            conf.reshape(B), attn_topk)
